# Initial kernel scaffold; baseline (speedup 1.0000x reference)
import jax
import jax.numpy as jnp
from jax.experimental import pallas as pl


def kernel(x, edge_index, batch, W1, b1, W2, b2, W3, b3, lin_W, lin_b):
    raise NotImplementedError("write your pallas kernel here")


# speedup vs baseline: 11.3948x; 11.3948x over previous
import functools

import jax
import jax.numpy as jnp
from jax import lax
from jax.experimental import pallas as pl
from jax.experimental.pallas import tpu as pltpu
from jax.experimental.pallas import tpu_sc as plsc

N = 10000
E = 160000
G = 64

NC = 2
NS = 16
NPT = N // NS

ECHUNK = 80
NCHUNK = (E // NS) // ECHUNK

DCHUNK = 40
DNCHUNK = (E // (NC * NS)) // DCHUNK
DEG_PAD = 10240

MBLK = 2000


def _sc_mesh():
  return plsc.VectorSubcoreMesh(core_axis_name="c", subcore_axis_name="s")


def _deg_sc(dstd):
  @functools.partial(
      pl.kernel,
      out_type=jax.ShapeDtypeStruct((NC, DEG_PAD), jnp.float32),
      mesh=_sc_mesh(),
      scratch_types=[
          pltpu.VMEM((DNCHUNK, DCHUNK), jnp.int32),
          pltpu.VMEM((640,), jnp.float32),
          pltpu.VMEM((48,), jnp.float32),
          pltpu.VMEM_SHARED((DEG_PAD,), jnp.float32),
      ],
  )
  def k(dst_hbm, deg_hbm, didx, init_v, ones_v, deg_sh):
    c = lax.axis_index("c")
    s = lax.axis_index("s")
    tid = c * NS + s
    iv = jnp.where(c == 0, 1.0, 0.0).astype(jnp.float32)

    @pl.loop(0, 640, step=16)
    def _(i):
      init_v[pl.ds(i, 16)] = jnp.full((16,), iv, jnp.float32)

    @pl.loop(0, 48, step=16)
    def _(i):
      ones_v[pl.ds(i, 16)] = jnp.ones((16,), jnp.float32)

    pltpu.sync_copy(dst_hbm.at[tid], didx)
    pltpu.sync_copy(init_v, deg_sh.at[pl.ds(s * 640, 640)])
    plsc.subcore_barrier()

    @pl.loop(0, DNCHUNK)
    def _(j):
      pltpu.sync_copy(ones_v.at[pl.ds(0, DCHUNK)], deg_sh.at[didx.at[j]],
                      add=True)

    plsc.subcore_barrier()
    pltpu.sync_copy(deg_sh.at[pl.ds(s * 640, 640)],
                    deg_hbm.at[c, pl.ds(s * 640, 640)])

  return k(dstd)


def _tc_first(x, W1, degA, degB):
  F = W1.shape[1]
  Fh = F // 2
  NB = N // MBLK

  def body(x_ref, w_ref, dga_ref, dgb_ref, h_ref, dinv_ref):
    deg = dga_ref[0, 0] + dgb_ref[0, 0]
    dinv = lax.rsqrt(deg)
    dinv_ref[0, 0] = dinv
    h = jnp.dot(x_ref[...], w_ref[...], preferred_element_type=jnp.float32)
    hs = h * dinv[:, None]
    h_ref[0] = hs[:, :Fh]
    h_ref[1] = hs[:, Fh:]

  h2, dinv3 = pl.pallas_call(
      body,
      grid=(NB,),
      in_specs=[
          pl.BlockSpec((MBLK, x.shape[1]), lambda i: (i, 0)),
          pl.BlockSpec(W1.shape, lambda i: (0, 0)),
          pl.BlockSpec((1, 1, MBLK), lambda i: (i, 0, 0)),
          pl.BlockSpec((1, 1, MBLK), lambda i: (i, 0, 0)),
      ],
      out_specs=[
          pl.BlockSpec((2, MBLK, Fh), lambda i: (0, i, 0)),
          pl.BlockSpec((1, 1, MBLK), lambda i: (i, 0, 0)),
      ],
      out_shape=[
          jax.ShapeDtypeStruct((2, N, Fh), jnp.float32),
          jax.ShapeDtypeStruct((NB, 1, MBLK), jnp.float32),
      ],
  )(x, W1, degA, degB)
  return h2.reshape(2 * N, Fh), dinv3


def _tc_mid(acc, dinv3, b, W):
  Fin = W.shape[0]
  Fo = W.shape[1]
  Foh = Fo // 2

  def body(acc_ref, dinv_ref, b_ref, w_ref, h_ref):
    z = jnp.concatenate([acc_ref[0], acc_ref[1]], axis=1)
    dv = dinv_ref[0, 0]
    z = jnp.maximum(z * dv[:, None] + b_ref[...][None, :], 0.0)
    h = jnp.dot(z, w_ref[...], preferred_element_type=jnp.float32)
    hs = h * dv[:, None]
    h_ref[0] = hs[:, :Foh]
    h_ref[1] = hs[:, Foh:]

  h2 = pl.pallas_call(
      body,
      grid=(N // MBLK,),
      in_specs=[
          pl.BlockSpec((2, MBLK, Fin // 2), lambda i: (0, i, 0)),
          pl.BlockSpec((1, 1, MBLK), lambda i: (i, 0, 0)),
          pl.BlockSpec((Fin,), lambda i: (0,)),
          pl.BlockSpec(W.shape, lambda i: (0, 0)),
      ],
      out_specs=pl.BlockSpec((2, MBLK, Foh), lambda i: (0, i, 0)),
      out_shape=jax.ShapeDtypeStruct((2, N, Foh), jnp.float32),
  )(acc, dinv3, b, W)
  return h2.reshape(2 * N, Foh)


def _msg_sc(h_flat, srcc, dst3, Fh):
  @functools.partial(
      pl.kernel,
      out_type=jax.ShapeDtypeStruct((2 * N, Fh), jnp.float32),
      mesh=_sc_mesh(),
      compiler_params=pltpu.CompilerParams(use_tc_tiling_on_sc=False),
      scratch_types=[
          pltpu.VMEM((NCHUNK, ECHUNK), jnp.int32),
          pltpu.VMEM((NCHUNK, ECHUNK), jnp.int32),
          pltpu.VMEM((ECHUNK, Fh), jnp.float32),
          pltpu.VMEM_SHARED((N, Fh), jnp.float32),
      ],
  )
  def k(h_hbm, src_hbm, dst_hbm, out_hbm, sidx, didx, rows, acc):
    c = lax.axis_index("c")
    s = lax.axis_index("s")
    pltpu.sync_copy(src_hbm.at[c, s], sidx)
    pltpu.sync_copy(dst_hbm.at[s], didx)
    pltpu.sync_copy(h_hbm.at[pl.ds(c * N + s * NPT, NPT)],
                    acc.at[pl.ds(s * NPT, NPT)])
    plsc.subcore_barrier()

    @pl.loop(0, NCHUNK)
    def _(j):
      pltpu.sync_copy(h_hbm.at[sidx.at[j]], rows)
      pltpu.sync_copy(rows, acc.at[didx.at[j]], add=True)

    plsc.subcore_barrier()
    pltpu.sync_copy(acc.at[pl.ds(s * NPT, NPT)],
                    out_hbm.at[pl.ds(c * N + s * NPT, NPT)])

  return k(h_flat, srcc, dst3)


def _msg_pool_sc(h_flat, srcc, dst3, dinv2, batch2):
  Fh = 32
  cp = pltpu.CompilerParams(needs_layout_passes=False,
                            use_tc_tiling_on_sc=False)

  @functools.partial(
      pl.kernel,
      out_type=jax.ShapeDtypeStruct((NC, NS, G * Fh), jnp.float32),
      mesh=_sc_mesh(),
      compiler_params=cp,
      scratch_types=[
          pltpu.VMEM((NCHUNK, ECHUNK), jnp.int32),
          pltpu.VMEM((NCHUNK, ECHUNK), jnp.int32),
          pltpu.VMEM((ECHUNK, Fh), jnp.float32),
          pltpu.VMEM((NPT, Fh), jnp.float32),
          pltpu.VMEM((640,), jnp.float32),
          pltpu.VMEM((640,), jnp.int32),
          pltpu.VMEM((G * Fh,), jnp.float32),
          pltpu.VMEM_SHARED((N, Fh), jnp.float32),
      ],
  )
  def k(h_hbm, src_hbm, dst_hbm, dinv_hbm, batch_hbm, pool_hbm,
        sidx, didx, rows, rslab, dvs, bts, pool, acc):
    c = lax.axis_index("c")
    s = lax.axis_index("s")
    pltpu.sync_copy(src_hbm.at[c, s], sidx)
    pltpu.sync_copy(dst_hbm.at[s], didx)
    pltpu.sync_copy(h_hbm.at[pl.ds(c * N + s * NPT, NPT)],
                    acc.at[pl.ds(s * NPT, NPT)])
    plsc.subcore_barrier()

    @pl.loop(0, NCHUNK)
    def _(j):
      pltpu.sync_copy(h_hbm.at[sidx.at[j]], rows)
      pltpu.sync_copy(rows, acc.at[didx.at[j]], add=True)

    plsc.subcore_barrier()

    pltpu.sync_copy(acc.at[pl.ds(s * NPT, NPT)], rslab)
    pltpu.sync_copy(dinv_hbm.at[s], dvs.at[pl.ds(0, NPT)])
    pltpu.sync_copy(batch_hbm.at[s], bts.at[pl.ds(0, NPT)])

    @pl.loop(0, G * Fh, step=16)
    def _(i):
      pool[pl.ds(i, 16)] = jnp.zeros((16,), jnp.float32)

    lanes = lax.iota(jnp.int32, 16)

    def do_row(row_i, b, dv):
      ri = jnp.full((16,), row_i, jnp.int32)
      base = lanes + b * Fh
      v0 = plsc.load_gather(rslab, [ri, lanes])
      v1 = plsc.load_gather(rslab, [ri, lanes + 16])
      plsc.addupdate_scatter(pool, [base], v0 * dv)
      plsc.addupdate_scatter(pool, [base + 16], v1 * dv)

    @pl.loop(0, NPT - 1, step=16)
    def _(i16):
      bvec = bts[pl.ds(i16, 16)]
      dvec = dvs[pl.ds(i16, 16)]
      for l in range(16):
        do_row(i16 + l, bvec[l], dvec[l])

    bvec = bts[pl.ds(NPT - 1, 16)]
    dvec = dvs[pl.ds(NPT - 1, 16)]
    do_row(NPT - 1, bvec[0], dvec[0])

    pltpu.sync_copy(pool, pool_hbm.at[c, s])

  return k(h_flat, srcc, dst3, dinv2, batch2)


def _head_tc(pool_part, batch, b3, lin_W, lin_b):
  def body(p_ref, batch_ref, b3_ref, w_ref, lb_ref, o_ref):
    p0 = jnp.sum(p_ref[0], axis=0)
    p1 = jnp.sum(p_ref[1], axis=0)
    gsum = jnp.concatenate([p0, p1], axis=1)
    gid = lax.broadcasted_iota(jnp.int32, (G, N), 0)
    onehot = (gid == batch_ref[...][None, :]).astype(jnp.float32)
    cnt = jnp.sum(onehot, axis=1)
    g = gsum / jnp.maximum(cnt, 1.0)[:, None] + b3_ref[...][None, :]
    o_ref[...] = (
        jnp.dot(g, w_ref[...], preferred_element_type=jnp.float32)
        + lb_ref[...][None, :])

  return pl.pallas_call(
      body,
      out_shape=jax.ShapeDtypeStruct((G, lin_W.shape[1]), jnp.float32),
  )(pool_part, batch, b3, lin_W, lin_b)


def kernel(x, edge_index, batch, W1, b1, W2, b2, W3, b3, lin_W, lin_b):
  src = edge_index[0].astype(jnp.int32)
  dst = edge_index[1].astype(jnp.int32)
  batch = batch.astype(jnp.int32)

  src3 = src.reshape(NS, NCHUNK, ECHUNK)
  srcc = jnp.stack([src3, src3 + N])
  dst3 = dst.reshape(NS, NCHUNK, ECHUNK)
  dstd = dst.reshape(NC * NS, DNCHUNK, DCHUNK)
  batch2 = batch.reshape(NS, NPT)

  deg2 = _deg_sc(dstd)
  degA = deg2[0, :N].reshape(N // MBLK, 1, MBLK)
  degB = deg2[1, :N].reshape(N // MBLK, 1, MBLK)
  h1, dinv3 = _tc_first(x, W1, degA, degB)
  acc1 = _msg_sc(h1, srcc, dst3, 128)
  h2 = _tc_mid(acc1.reshape(2, N, 128), dinv3, b1, W2)
  acc2 = _msg_sc(h2, srcc, dst3, 64)
  h3 = _tc_mid(acc2.reshape(2, N, 64), dinv3, b2, W3)
  dinv2 = dinv3.reshape(NS, NPT)
  pool_part = _msg_pool_sc(h3, srcc, dst3, dinv2, batch2)
  pool_part = pool_part.reshape(NC, NS, G, 32)
  return _head_tc(pool_part, batch, b3, lin_W, lin_b)


# speedup vs baseline: 17.5702x; 1.5419x over previous
import functools

import jax
import jax.numpy as jnp
from jax import lax
from jax.experimental import pallas as pl
from jax.experimental.pallas import tpu as pltpu
from jax.experimental.pallas import tpu_sc as plsc

N = 10000
E = 160000
G = 64

NC = 2
NS = 16
NPT = N // NS

ECHUNK = 80
NCHUNK = (E // NS) // ECHUNK

DCHUNK = 40
DNCHUNK = (E // (NC * NS)) // DCHUNK
DEG_PAD = 10240

MBLK = 2000


def _sc_mesh():
  return plsc.VectorSubcoreMesh(core_axis_name="c", subcore_axis_name="s")


def _deg_sc(dstd):
  @functools.partial(
      pl.kernel,
      out_type=jax.ShapeDtypeStruct((NC, DEG_PAD), jnp.float32),
      mesh=_sc_mesh(),
      scratch_types=[
          pltpu.VMEM((DNCHUNK, DCHUNK), jnp.int32),
          pltpu.VMEM((640,), jnp.float32),
          pltpu.VMEM((48,), jnp.float32),
          pltpu.VMEM_SHARED((DEG_PAD,), jnp.float32),
      ],
  )
  def k(dst_hbm, deg_hbm, didx, init_v, ones_v, deg_sh):
    c = lax.axis_index("c")
    s = lax.axis_index("s")
    tid = c * NS + s
    iv = jnp.where(c == 0, 1.0, 0.0).astype(jnp.float32)

    @pl.loop(0, 640, step=16)
    def _(i):
      init_v[pl.ds(i, 16)] = jnp.full((16,), iv, jnp.float32)

    @pl.loop(0, 48, step=16)
    def _(i):
      ones_v[pl.ds(i, 16)] = jnp.ones((16,), jnp.float32)

    pltpu.sync_copy(dst_hbm.at[tid], didx)
    pltpu.sync_copy(init_v, deg_sh.at[pl.ds(s * 640, 640)])
    plsc.subcore_barrier()

    @pl.loop(0, DNCHUNK)
    def _(j):
      pltpu.sync_copy(ones_v.at[pl.ds(0, DCHUNK)], deg_sh.at[didx.at[j]],
                      add=True)

    plsc.subcore_barrier()
    pltpu.sync_copy(deg_sh.at[pl.ds(s * 640, 640)],
                    deg_hbm.at[c, pl.ds(s * 640, 640)])

  return k(dstd)


def _tc_first(x, W1, degA, degB):
  F = W1.shape[1]
  Fh = F // 2
  NB = N // MBLK

  def body(x_ref, w_ref, dga_ref, dgb_ref, h_ref, dinv_ref):
    deg = dga_ref[0, 0] + dgb_ref[0, 0]
    dinv = lax.rsqrt(deg)
    dinv_ref[0, 0] = dinv
    h = jnp.dot(x_ref[...], w_ref[...], preferred_element_type=jnp.float32)
    hs = h * dinv[:, None]
    h_ref[0] = hs[:, :Fh]
    h_ref[1] = hs[:, Fh:]

  h2, dinv3 = pl.pallas_call(
      body,
      grid=(NB,),
      in_specs=[
          pl.BlockSpec((MBLK, x.shape[1]), lambda i: (i, 0)),
          pl.BlockSpec(W1.shape, lambda i: (0, 0)),
          pl.BlockSpec((1, 1, MBLK), lambda i: (i, 0, 0)),
          pl.BlockSpec((1, 1, MBLK), lambda i: (i, 0, 0)),
      ],
      out_specs=[
          pl.BlockSpec((2, MBLK, Fh), lambda i: (0, i, 0)),
          pl.BlockSpec((1, 1, MBLK), lambda i: (i, 0, 0)),
      ],
      out_shape=[
          jax.ShapeDtypeStruct((2, N, Fh), jnp.float32),
          jax.ShapeDtypeStruct((NB, 1, MBLK), jnp.float32),
      ],
  )(x, W1, degA, degB)
  return h2.reshape(2 * N, Fh), dinv3


def _tc_mid(acc, dinv3, b, W):
  Fin = W.shape[0]
  Fo = W.shape[1]
  Foh = Fo // 2

  def body(acc_ref, dinv_ref, b_ref, w_ref, h_ref):
    z = jnp.concatenate([acc_ref[0], acc_ref[1]], axis=1)
    dv = dinv_ref[0, 0]
    z = jnp.maximum(z * dv[:, None] + b_ref[...][None, :], 0.0)
    h = jnp.dot(z, w_ref[...], preferred_element_type=jnp.float32)
    hs = h * dv[:, None]
    h_ref[0] = hs[:, :Foh]
    h_ref[1] = hs[:, Foh:]

  h2 = pl.pallas_call(
      body,
      grid=(N // MBLK,),
      in_specs=[
          pl.BlockSpec((2, MBLK, Fin // 2), lambda i: (0, i, 0)),
          pl.BlockSpec((1, 1, MBLK), lambda i: (i, 0, 0)),
          pl.BlockSpec((Fin,), lambda i: (0,)),
          pl.BlockSpec(W.shape, lambda i: (0, 0)),
      ],
      out_specs=pl.BlockSpec((2, MBLK, Foh), lambda i: (0, i, 0)),
      out_shape=jax.ShapeDtypeStruct((2, N, Foh), jnp.float32),
  )(acc, dinv3, b, W)
  return h2.reshape(2 * N, Foh)


def _edge_loop(h_hbm, sidx, didx, rows0, rows1, acc, sem0, sem1):
  pltpu.async_copy(h_hbm.at[sidx.at[0]], rows0, sem0)

  @pl.loop(0, NCHUNK - 1, step=2)
  def _(j):
    pltpu.async_copy(h_hbm.at[sidx.at[j + 1]], rows1, sem1)
    pltpu.make_async_copy(h_hbm.at[sidx.at[j]], rows0, sem0).wait()
    pltpu.sync_copy(rows0, acc.at[didx.at[j]], add=True)
    pltpu.async_copy(h_hbm.at[sidx.at[j + 2]], rows0, sem0)
    pltpu.make_async_copy(h_hbm.at[sidx.at[j + 1]], rows1, sem1).wait()
    pltpu.sync_copy(rows1, acc.at[didx.at[j + 1]], add=True)

  pltpu.make_async_copy(h_hbm.at[sidx.at[NCHUNK - 1]], rows0, sem0).wait()
  pltpu.sync_copy(rows0, acc.at[didx.at[NCHUNK - 1]], add=True)


def _msg_sc(h_flat, srcc, dst3, Fh):
  @functools.partial(
      pl.kernel,
      out_type=jax.ShapeDtypeStruct((2 * N, Fh), jnp.float32),
      mesh=_sc_mesh(),
      compiler_params=pltpu.CompilerParams(use_tc_tiling_on_sc=False),
      scratch_types=[
          pltpu.VMEM((NCHUNK, ECHUNK), jnp.int32),
          pltpu.VMEM((NCHUNK, ECHUNK), jnp.int32),
          pltpu.VMEM((ECHUNK, Fh), jnp.float32),
          pltpu.VMEM((ECHUNK, Fh), jnp.float32),
          pltpu.VMEM_SHARED((N, Fh), jnp.float32),
          pltpu.SemaphoreType.DMA,
          pltpu.SemaphoreType.DMA,
      ],
  )
  def k(h_hbm, src_hbm, dst_hbm, out_hbm, sidx, didx, rows0, rows1, acc,
        sem0, sem1):
    c = lax.axis_index("c")
    s = lax.axis_index("s")
    pltpu.sync_copy(src_hbm.at[c, s], sidx)
    pltpu.sync_copy(dst_hbm.at[s], didx)
    pltpu.sync_copy(h_hbm.at[pl.ds(c * N + s * NPT, NPT)],
                    acc.at[pl.ds(s * NPT, NPT)])
    plsc.subcore_barrier()
    _edge_loop(h_hbm, sidx, didx, rows0, rows1, acc, sem0, sem1)
    plsc.subcore_barrier()
    pltpu.sync_copy(acc.at[pl.ds(s * NPT, NPT)],
                    out_hbm.at[pl.ds(c * N + s * NPT, NPT)])

  return k(h_flat, srcc, dst3)


def _msg_pool_sc(h_flat, srcc, dst3, dinv2, batch2):
  Fh = 32
  cp = pltpu.CompilerParams(needs_layout_passes=False,
                            use_tc_tiling_on_sc=False)

  @functools.partial(
      pl.kernel,
      out_type=jax.ShapeDtypeStruct((NC, NS, G * Fh), jnp.float32),
      mesh=_sc_mesh(),
      compiler_params=cp,
      scratch_types=[
          pltpu.VMEM((NCHUNK, ECHUNK), jnp.int32),
          pltpu.VMEM((NCHUNK, ECHUNK), jnp.int32),
          pltpu.VMEM((ECHUNK, Fh), jnp.float32),
          pltpu.VMEM((ECHUNK, Fh), jnp.float32),
          pltpu.VMEM((NPT, Fh), jnp.float32),
          pltpu.VMEM((640,), jnp.float32),
          pltpu.VMEM((640,), jnp.int32),
          pltpu.VMEM((G * Fh,), jnp.float32),
          pltpu.VMEM_SHARED((N, Fh), jnp.float32),
          pltpu.SemaphoreType.DMA,
          pltpu.SemaphoreType.DMA,
      ],
  )
  def k(h_hbm, src_hbm, dst_hbm, dinv_hbm, batch_hbm, pool_hbm,
        sidx, didx, rows0, rows1, rslab, dvs, bts, pool, acc, sem0, sem1):
    c = lax.axis_index("c")
    s = lax.axis_index("s")
    pltpu.sync_copy(src_hbm.at[c, s], sidx)
    pltpu.sync_copy(dst_hbm.at[s], didx)
    pltpu.sync_copy(h_hbm.at[pl.ds(c * N + s * NPT, NPT)],
                    acc.at[pl.ds(s * NPT, NPT)])
    plsc.subcore_barrier()
    _edge_loop(h_hbm, sidx, didx, rows0, rows1, acc, sem0, sem1)
    plsc.subcore_barrier()

    pltpu.sync_copy(acc.at[pl.ds(s * NPT, NPT)], rslab)
    pltpu.sync_copy(dinv_hbm.at[s], dvs.at[pl.ds(0, NPT)])
    pltpu.sync_copy(batch_hbm.at[s], bts.at[pl.ds(0, NPT)])

    @pl.loop(0, G * Fh, step=16)
    def _(i):
      pool[pl.ds(i, 16)] = jnp.zeros((16,), jnp.float32)

    lanes = lax.iota(jnp.int32, 16)

    def do_row(row_i, b, dv):
      ri = jnp.full((16,), row_i, jnp.int32)
      base = lanes + b * Fh
      v0 = plsc.load_gather(rslab, [ri, lanes])
      v1 = plsc.load_gather(rslab, [ri, lanes + 16])
      plsc.addupdate_scatter(pool, [base], v0 * dv)
      plsc.addupdate_scatter(pool, [base + 16], v1 * dv)

    @pl.loop(0, NPT - 1, step=16)
    def _(i16):
      bvec = bts[pl.ds(i16, 16)]
      dvec = dvs[pl.ds(i16, 16)]
      for l in range(16):
        do_row(i16 + l, bvec[l], dvec[l])

    bvec = bts[pl.ds(NPT - 1, 16)]
    dvec = dvs[pl.ds(NPT - 1, 16)]
    do_row(NPT - 1, bvec[0], dvec[0])

    pltpu.sync_copy(pool, pool_hbm.at[c, s])

  return k(h_flat, srcc, dst3, dinv2, batch2)


def _head_tc(pool_part, batch, b3, lin_W, lin_b):
  def body(p_ref, batch_ref, b3_ref, w_ref, lb_ref, o_ref):
    p0 = jnp.sum(p_ref[0], axis=0)
    p1 = jnp.sum(p_ref[1], axis=0)
    gsum = jnp.concatenate([p0, p1], axis=1)
    gid = lax.broadcasted_iota(jnp.int32, (G, N), 0)
    onehot = (gid == batch_ref[...][None, :]).astype(jnp.float32)
    cnt = jnp.sum(onehot, axis=1)
    g = gsum / jnp.maximum(cnt, 1.0)[:, None] + b3_ref[...][None, :]
    o_ref[...] = (
        jnp.dot(g, w_ref[...], preferred_element_type=jnp.float32)
        + lb_ref[...][None, :])

  return pl.pallas_call(
      body,
      out_shape=jax.ShapeDtypeStruct((G, lin_W.shape[1]), jnp.float32),
  )(pool_part, batch, b3, lin_W, lin_b)


def kernel(x, edge_index, batch, W1, b1, W2, b2, W3, b3, lin_W, lin_b):
  src = edge_index[0].astype(jnp.int32)
  dst = edge_index[1].astype(jnp.int32)
  batch = batch.astype(jnp.int32)

  src3 = src.reshape(NS, NCHUNK, ECHUNK)
  srcc = jnp.stack([src3, src3 + N])
  dst3 = dst.reshape(NS, NCHUNK, ECHUNK)
  dstd = dst.reshape(NC * NS, DNCHUNK, DCHUNK)
  batch2 = batch.reshape(NS, NPT)

  deg2 = _deg_sc(dstd)
  degA = deg2[0, :N].reshape(N // MBLK, 1, MBLK)
  degB = deg2[1, :N].reshape(N // MBLK, 1, MBLK)
  h1, dinv3 = _tc_first(x, W1, degA, degB)
  acc1 = _msg_sc(h1, srcc, dst3, 128)
  h2 = _tc_mid(acc1.reshape(2, N, 128), dinv3, b1, W2)
  acc2 = _msg_sc(h2, srcc, dst3, 64)
  h3 = _tc_mid(acc2.reshape(2, N, 64), dinv3, b2, W3)
  dinv2 = dinv3.reshape(NS, NPT)
  pool_part = _msg_pool_sc(h3, srcc, dst3, dinv2, batch2)
  pool_part = pool_part.reshape(NC, NS, G, 32)
  return _head_tc(pool_part, batch, b3, lin_W, lin_b)
